# baseline (speedup 1.0000x reference)
import functools

import jax
import jax.numpy as jnp
from jax.experimental import pallas as pl
from jax.experimental.pallas import tpu as pltpu


def _dot(a, b):
    return jax.lax.dot_general(
        a, b, (((1,), (0,)), ((), ())),
        precision=jax.lax.Precision.DEFAULT,
        preferred_element_type=jnp.float32,
    )


def _pass1_body(x_ref, w1_ref, b1_ref, w2_ref, adj_ref, adjbf_ref, s2_ref,
                s1_ref):
    s = pl.program_id(0)

    @pl.when(s == 0)
    def _():
        s1_ref[...] = _dot(x_ref[...], w1_ref[...]).astype(jnp.bfloat16)

    @pl.when(s >= 1)
    def _():
        a = adj_ref[...].astype(jnp.bfloat16)
        adjbf_ref[...] = a
        h = _dot(a, s1_ref[...])
        h = jnp.maximum(h + b1_ref[...], 0.0).astype(jnp.bfloat16)
        s2_ref[...] = _dot(h, w2_ref[...]).astype(jnp.bfloat16)


def _pass2_body(adjbf_ref, s2_ref, b2_ref, out_ref):
    o = _dot(adjbf_ref[...], s2_ref[...]) + b2_ref[...]
    m = jnp.max(o, axis=1, keepdims=True)
    lse = m + jnp.log(jnp.sum(jnp.exp(o - m), axis=1, keepdims=True))
    out_ref[...] = o - lse


def kernel(x, adj, layer_dropout, stage1_flag, W1, b1, W2, b2):
    n, nfeat = x.shape
    nhid = W1.shape[1]
    nclass = W2.shape[1]

    bm = 400
    tiles = n // bm

    def head_map(s):
        return (jnp.maximum(s - 1, 0), 0)

    adjbf, s2 = pl.pallas_call(
        _pass1_body,
        grid=(1 + tiles,),
        in_specs=[
            pl.BlockSpec((n, nfeat), lambda s: (0, 0)),
            pl.BlockSpec((nfeat, nhid), lambda s: (0, 0)),
            pl.BlockSpec((1, nhid), lambda s: (0, 0)),
            pl.BlockSpec((nhid, nclass), lambda s: (0, 0)),
            pl.BlockSpec((bm, n), head_map),
        ],
        out_specs=[
            pl.BlockSpec((bm, n), head_map),
            pl.BlockSpec((bm, nclass), head_map),
        ],
        out_shape=[
            jax.ShapeDtypeStruct((n, n), jnp.bfloat16),
            jax.ShapeDtypeStruct((n, nclass), jnp.bfloat16),
        ],
        scratch_shapes=[pltpu.VMEM((n, nhid), jnp.bfloat16)],
        compiler_params=pltpu.CompilerParams(
            vmem_limit_bytes=110 * 1024 * 1024),
    )(x, W1, b1.reshape(1, nhid), W2.astype(jnp.bfloat16), adj)

    logp = pl.pallas_call(
        _pass2_body,
        grid=(tiles,),
        in_specs=[
            pl.BlockSpec((bm, n), lambda s: (s, 0)),
            pl.BlockSpec((n, nclass), lambda s: (0, 0)),
            pl.BlockSpec((1, nclass), lambda s: (0, 0)),
        ],
        out_specs=pl.BlockSpec((bm, nclass), lambda s: (s, 0)),
        out_shape=jax.ShapeDtypeStruct((n, nclass), jnp.float32),
    )(adjbf, s2, b2.reshape(1, nclass))

    node_lastlayer = jnp.ones((n, 1), dtype=jnp.float32)
    return (logp, node_lastlayer)


# speedup vs baseline: 1.0958x; 1.0958x over previous
import functools

import jax
import jax.numpy as jnp
from jax.experimental import pallas as pl
from jax.experimental.pallas import tpu as pltpu


def _dot(a, b):
    return jax.lax.dot_general(
        a, b, (((1,), (0,)), ((), ())),
        precision=jax.lax.Precision.DEFAULT,
        preferred_element_type=jnp.float32,
    )


def _body(x_ref, w1_ref, b1_ref, w2_ref, b2_ref, adj_ref, out_ref,
          xbf_ref, s2_ref, *, bm, tiles):
    s = pl.program_id(0)

    @pl.when(s == 0)
    def _():
        xbf_ref[...] = x_ref[...].astype(jnp.bfloat16)

    @pl.when(s < tiles)
    def _():
        a = adj_ref[...].astype(jnp.bfloat16)
        t = _dot(a, xbf_ref[...])
        h = _dot(t.astype(jnp.bfloat16), w1_ref[...]) + b1_ref[...]
        h = jnp.maximum(h, 0.0).astype(jnp.bfloat16)
        s2_ref[pl.ds(s * bm, bm), :] = _dot(h, w2_ref[...]).astype(
            jnp.bfloat16)

    @pl.when(s >= tiles)
    def _():
        a = adj_ref[...].astype(jnp.bfloat16)
        o = _dot(a, s2_ref[...]) + b2_ref[...]
        m = jnp.max(o, axis=1, keepdims=True)
        lse = m + jnp.log(jnp.sum(jnp.exp(o - m), axis=1, keepdims=True))
        out_ref[...] = o - lse


def kernel(x, adj, layer_dropout, stage1_flag, W1, b1, W2, b2):
    n, nfeat = x.shape
    nhid = W1.shape[1]
    nclass = W2.shape[1]

    bm = 400
    tiles = n // bm
    last = 2 * tiles - 1

    def adj_map(s):
        return (jnp.where(s < tiles, s, last - s), 0)

    def out_map(s):
        return (jnp.where(s < tiles, tiles - 1, last - s), 0)

    body = functools.partial(_body, bm=bm, tiles=tiles)

    logp = pl.pallas_call(
        body,
        grid=(2 * tiles,),
        in_specs=[
            pl.BlockSpec((n, nfeat), lambda s: (0, 0)),
            pl.BlockSpec((nfeat, nhid), lambda s: (0, 0)),
            pl.BlockSpec((1, nhid), lambda s: (0, 0)),
            pl.BlockSpec((nhid, nclass), lambda s: (0, 0)),
            pl.BlockSpec((1, nclass), lambda s: (0, 0)),
            pl.BlockSpec((bm, n), adj_map),
        ],
        out_specs=pl.BlockSpec((bm, nclass), out_map),
        out_shape=jax.ShapeDtypeStruct((n, nclass), jnp.float32),
        scratch_shapes=[
            pltpu.VMEM((n, nfeat), jnp.bfloat16),
            pltpu.VMEM((n, nclass), jnp.bfloat16),
        ],
    )(x, W1.astype(jnp.bfloat16), b1.reshape(1, nhid),
      W2.astype(jnp.bfloat16), b2.reshape(1, nclass), adj)

    node_lastlayer = jnp.ones((n, 1), dtype=jnp.float32)
    return (logp, node_lastlayer)


# speedup vs baseline: 1.1042x; 1.0077x over previous
import functools

import jax
import jax.numpy as jnp
from jax.experimental import pallas as pl
from jax.experimental.pallas import tpu as pltpu


def _dot(a, b):
    return jax.lax.dot_general(
        a, b, (((1,), (0,)), ((), ())),
        precision=jax.lax.Precision.DEFAULT,
        preferred_element_type=jnp.float32,
    )


def _body(x_ref, w1_ref, b1_ref, w2_ref, b2_ref, adj_ref, out_ref,
          xbf_ref, s2_ref, *, bm, tiles):
    s = pl.program_id(0)

    @pl.when(s == 0)
    def _():
        xbf_ref[...] = x_ref[...]

    @pl.when(s < tiles)
    def _():
        t = _dot(adj_ref[...], xbf_ref[...])
        h = _dot(t, w1_ref[...]) + b1_ref[...]
        h = jnp.maximum(h, 0.0)
        s2_ref[pl.ds(s * bm, bm), :] = _dot(h, w2_ref[...])

    @pl.when(s >= tiles)
    def _():
        o = _dot(adj_ref[...], s2_ref[...]) + b2_ref[...]
        m = jnp.max(o, axis=1, keepdims=True)
        lse = m + jnp.log(jnp.sum(jnp.exp(o - m), axis=1, keepdims=True))
        out_ref[...] = o - lse


def kernel(x, adj, layer_dropout, stage1_flag, W1, b1, W2, b2):
    n, nfeat = x.shape
    nhid = W1.shape[1]
    nclass = W2.shape[1]

    bm = 400
    tiles = n // bm
    last = 2 * tiles - 1

    def adj_map(s):
        return (jnp.where(s < tiles, s, last - s), 0)

    def out_map(s):
        return (jnp.where(s < tiles, tiles - 1, last - s), 0)

    body = functools.partial(_body, bm=bm, tiles=tiles)

    logp = pl.pallas_call(
        body,
        grid=(2 * tiles,),
        in_specs=[
            pl.BlockSpec((n, nfeat), lambda s: (0, 0)),
            pl.BlockSpec((nfeat, nhid), lambda s: (0, 0)),
            pl.BlockSpec((1, nhid), lambda s: (0, 0)),
            pl.BlockSpec((nhid, nclass), lambda s: (0, 0)),
            pl.BlockSpec((1, nclass), lambda s: (0, 0)),
            pl.BlockSpec((bm, n), adj_map),
        ],
        out_specs=pl.BlockSpec((bm, nclass), out_map),
        out_shape=jax.ShapeDtypeStruct((n, nclass), jnp.float32),
        scratch_shapes=[
            pltpu.VMEM((n, nfeat), jnp.float32),
            pltpu.VMEM((n, nclass), jnp.float32),
        ],
    )(x, W1, b1.reshape(1, nhid), W2, b2.reshape(1, nclass), adj)

    node_lastlayer = jnp.ones((n, 1), dtype=jnp.float32)
    return (logp, node_lastlayer)
